# baseline (speedup 1.0000x reference)
import functools

import jax
import jax.numpy as jnp
from jax.experimental import pallas as pl
from jax.experimental.pallas import tpu as pltpu
from jax.experimental.pallas import tpu_sc as plsc

_N = 10000
_NW = 32
_RPW = 320
_LAST_ROWS = _N - (_NW - 1) * _RPW
_SUB = 4
_L = 16


def _v_row_kernel(x_ref, wt_ref, b_ref, v_ref):
    wbar_row = jnp.mean(wt_ref[...], axis=0, keepdims=True)
    bbar = jnp.mean(b_ref[...])
    v_ref[...] = jax.lax.dot_general(
        wbar_row, x_ref[...],
        (((1,), (1,)), ((), ())),
        preferred_element_type=jnp.float32,
        precision=jax.lax.Precision.HIGHEST,
    ) + bbar


def _sc_matvec_body(adj_hbm, v_hbm, out_hbm,
                    v_vmem, buf, outv, sem0, sem1):
    c = jax.lax.axis_index("c")
    s = jax.lax.axis_index("s")
    wid = s * 2 + c
    base = wid * _RPW
    is_last = wid == _NW - 1
    nrows = jnp.where(is_last, _LAST_ROWS, _RPW)
    nsub = nrows // _SUB
    nsuper = nrows // _L

    pltpu.sync_copy(v_hbm, v_vmem)

    sems = (sem0, sem1)

    def dma(k, slot):
        return pltpu.make_async_copy(
            adj_hbm.at[pl.ds(base + k * _SUB, _SUB)], buf.at[slot],
            sems[slot])

    dma(0, 0).start()

    nchunk = _N // (_L * 5)

    def super_body(sb, carry):
        accs = []
        for q in range(_L // _SUB):
            k = sb * (_L // _SUB) + q
            slot = q % 2
            dma(k, slot).wait()

            @pl.when(k + 1 < nsub)
            def _():
                dma(k + 1, (q + 1) % 2).start()

            def chunk_body(j, acc4):
                a0, a1, a2, a3 = acc4
                off = j * (_L * 5)
                for u in range(5):
                    o = off + u * _L
                    vc = v_vmem[pl.ds(o, _L)]
                    a0 = a0 + buf[slot, 0, pl.ds(o, _L)] * vc
                    a1 = a1 + buf[slot, 1, pl.ds(o, _L)] * vc
                    a2 = a2 + buf[slot, 2, pl.ds(o, _L)] * vc
                    a3 = a3 + buf[slot, 3, pl.ds(o, _L)] * vc
                return a0, a1, a2, a3

            z = jnp.zeros((_L,), jnp.float32)
            accs.extend(jax.lax.fori_loop(0, nchunk, chunk_body,
                                          (z, z, z, z)))
        lane_ids = jax.lax.iota(jnp.int32, _L)
        res = jnp.zeros((_L,), jnp.float32)
        for r, acc in enumerate(accs):
            res = jnp.where(lane_ids == r, jnp.sum(acc), res)
        outv[pl.ds(sb * _L, _L)] = res
        return carry

    jax.lax.fori_loop(0, nsuper, super_body, 0)

    @pl.when(jnp.logical_not(is_last))
    def _():
        pltpu.sync_copy(outv, out_hbm.at[pl.ds(base, _RPW)])

    @pl.when(is_last)
    def _():
        pltpu.sync_copy(outv.at[pl.ds(0, _LAST_ROWS)],
                        out_hbm.at[pl.ds(base, _LAST_ROWS)])


_sc_matvec = pl.kernel(
    _sc_matvec_body,
    out_type=jax.ShapeDtypeStruct((_N,), jnp.float32),
    mesh=plsc.VectorSubcoreMesh(core_axis_name="c", subcore_axis_name="s"),
    compiler_params=pltpu.CompilerParams(needs_layout_passes=False),
    scratch_types=[
        pltpu.VMEM((_N,), jnp.float32),
        pltpu.VMEM((2, _SUB, _N), jnp.float32),
        pltpu.VMEM((_RPW,), jnp.float32),
        pltpu.SemaphoreType.DMA,
        pltpu.SemaphoreType.DMA,
    ],
)


def kernel(x, adj, W, b):
    n, d = x.shape
    c = W.shape[1]
    vrow = pl.pallas_call(
        _v_row_kernel,
        out_shape=jax.ShapeDtypeStruct((1, n), jnp.float32),
    )(x, W.T, b.reshape(1, c))
    return _sc_matvec(adj, vrow.reshape(n))


# speedup vs baseline: 1.3925x; 1.3925x over previous
import jax
import jax.numpy as jnp
from jax.experimental import pallas as pl
from jax.experimental.pallas import tpu as pltpu
from jax.experimental.pallas import tpu_sc as plsc

_N = 10000
_BN_TC = 400
_S0 = 6400
_NSC = _N - _S0
_NW = 32
_L = 16
_SUB = 4
_TSUP = _NSC // _L
_BSUP = _TSUP // _NW
_XTRA = _TSUP % _NW
_MAXR = _L * (_BSUP + (1 if _XTRA else 0))


def _v_row_kernel(x_ref, wt_ref, b_ref, v_ref):
    wbar_row = jnp.mean(wt_ref[...], axis=0, keepdims=True)
    bbar = jnp.mean(b_ref[...])
    v_ref[...] = jax.lax.dot_general(
        wbar_row, x_ref[...],
        (((1,), (1,)), ((), ())),
        preferred_element_type=jnp.float32,
        precision=jax.lax.Precision.HIGHEST,
    ) + bbar


def _tc_matvec_kernel(adj_ref, vrow_ref, out_ref):
    out_ref[...] = jnp.sum(adj_ref[...] * vrow_ref[...], axis=1,
                           keepdims=True)


def _sc_matvec_body(adj_hbm, v_hbm, out_hbm, v_vmem, buf, outv, sem0, sem1):
    c = jax.lax.axis_index("c")
    s = jax.lax.axis_index("s")
    wid = s * 2 + c
    nsuper = _BSUP + jnp.where(wid < _XTRA, 1, 0)
    supers_before = _BSUP * wid + jnp.minimum(wid, _XTRA)
    base = _S0 + _L * supers_before
    obase = _L * supers_before
    nsub = nsuper * (_L // _SUB)

    pltpu.sync_copy(v_hbm, v_vmem)

    sems = (sem0, sem1)

    def dma(k, slot):
        return pltpu.make_async_copy(
            adj_hbm.at[pl.ds(base + k * _SUB, _SUB)], buf.at[slot],
            sems[slot])

    dma(0, 0).start()

    nchunk = _N // (_L * 5)

    def super_body(sb, carry):
        accs = []
        for q in range(_L // _SUB):
            k = sb * (_L // _SUB) + q
            slot = q % 2
            dma(k, slot).wait()

            @pl.when(k + 1 < nsub)
            def _():
                dma(k + 1, (q + 1) % 2).start()

            def chunk_body(j, acc4):
                a0, a1, a2, a3 = acc4
                off = j * (_L * 5)
                for u in range(5):
                    o = off + u * _L
                    vc = v_vmem[pl.ds(o, _L)]
                    a0 = a0 + buf[slot, 0, pl.ds(o, _L)] * vc
                    a1 = a1 + buf[slot, 1, pl.ds(o, _L)] * vc
                    a2 = a2 + buf[slot, 2, pl.ds(o, _L)] * vc
                    a3 = a3 + buf[slot, 3, pl.ds(o, _L)] * vc
                return a0, a1, a2, a3

            z = jnp.zeros((_L,), jnp.float32)
            accs.extend(jax.lax.fori_loop(0, nchunk, chunk_body,
                                          (z, z, z, z)))
        lane_ids = jax.lax.iota(jnp.int32, _L)
        res = jnp.zeros((_L,), jnp.float32)
        for r, acc in enumerate(accs):
            res = jnp.where(lane_ids == r, jnp.sum(acc), res)
        outv[pl.ds(sb * _L, _L)] = res
        return carry

    jax.lax.fori_loop(0, nsuper, super_body, 0)

    if _XTRA:
        @pl.when(wid < _XTRA)
        def _():
            pltpu.sync_copy(outv, out_hbm.at[pl.ds(obase, _MAXR)])

        @pl.when(wid >= _XTRA)
        def _():
            pltpu.sync_copy(outv.at[pl.ds(0, _L * _BSUP)],
                            out_hbm.at[pl.ds(obase, _L * _BSUP)])
    else:
        pltpu.sync_copy(outv, out_hbm.at[pl.ds(obase, _MAXR)])


_sc_matvec = pl.kernel(
    _sc_matvec_body,
    out_type=jax.ShapeDtypeStruct((_NSC,), jnp.float32),
    mesh=plsc.VectorSubcoreMesh(core_axis_name="c", subcore_axis_name="s"),
    compiler_params=pltpu.CompilerParams(needs_layout_passes=False),
    scratch_types=[
        pltpu.VMEM((_N,), jnp.float32),
        pltpu.VMEM((2, _SUB, _N), jnp.float32),
        pltpu.VMEM((_MAXR,), jnp.float32),
        pltpu.SemaphoreType.DMA,
        pltpu.SemaphoreType.DMA,
    ],
)


def kernel(x, adj, W, b):
    n, d = x.shape
    c = W.shape[1]
    vrow = pl.pallas_call(
        _v_row_kernel,
        out_shape=jax.ShapeDtypeStruct((1, n), jnp.float32),
    )(x, W.T, b.reshape(1, c))

    out_tc = pl.pallas_call(
        _tc_matvec_kernel,
        grid=(_S0 // _BN_TC,),
        in_specs=[
            pl.BlockSpec((_BN_TC, n), lambda i: (i, 0)),
            pl.BlockSpec((1, n), lambda i: (0, 0)),
        ],
        out_specs=pl.BlockSpec((_BN_TC, 1), lambda i: (i, 0)),
        out_shape=jax.ShapeDtypeStruct((_S0, 1), jnp.float32),
    )(adj, vrow)

    out_sc = _sc_matvec(adj, vrow.reshape(n))
    return jnp.concatenate([out_tc[:, 0], out_sc])


# speedup vs baseline: 1.3987x; 1.0044x over previous
import jax
import jax.numpy as jnp
from jax.experimental import pallas as pl
from jax.experimental.pallas import tpu as pltpu
from jax.experimental.pallas import tpu_sc as plsc

_N = 10000
_BN_TC = 400
_S0 = 8000
_NSC = _N - _S0
_NW = 32
_L = 16
_SUB = 4
_TSUP = _NSC // _L
_BSUP = _TSUP // _NW
_XTRA = _TSUP % _NW
_MAXR = _L * (_BSUP + (1 if _XTRA else 0))


def _v_row_kernel(x_ref, wt_ref, b_ref, v_ref):
    wbar_row = jnp.mean(wt_ref[...], axis=0, keepdims=True)
    bbar = jnp.mean(b_ref[...])
    v_ref[...] = jax.lax.dot_general(
        wbar_row, x_ref[...],
        (((1,), (1,)), ((), ())),
        preferred_element_type=jnp.float32,
        precision=jax.lax.Precision.HIGHEST,
    ) + bbar


def _tc_matvec_kernel(adj_ref, vrow_ref, out_ref):
    out_ref[...] = jnp.sum(adj_ref[...] * vrow_ref[...], axis=1,
                           keepdims=True)


def _sc_matvec_body(adj_hbm, v_hbm, out_hbm, v_vmem, buf, outv, sem0, sem1):
    c = jax.lax.axis_index("c")
    s = jax.lax.axis_index("s")
    wid = s * 2 + c
    nsuper = _BSUP + jnp.where(wid < _XTRA, 1, 0)
    supers_before = _BSUP * wid + jnp.minimum(wid, _XTRA)
    base = _S0 + _L * supers_before
    obase = _L * supers_before
    nsub = nsuper * (_L // _SUB)

    pltpu.sync_copy(v_hbm, v_vmem)

    sems = (sem0, sem1)

    def dma(k, slot):
        return pltpu.make_async_copy(
            adj_hbm.at[pl.ds(base + k * _SUB, _SUB)], buf.at[slot],
            sems[slot])

    dma(0, 0).start()

    nchunk = _N // _L

    def super_body(sb, carry):
        accs = []
        for q in range(_L // _SUB):
            k = sb * (_L // _SUB) + q
            slot = q % 2
            dma(k, slot).wait()

            @pl.when(k + 1 < nsub)
            def _():
                dma(k + 1, (q + 1) % 2).start()

            z = jnp.zeros((_L,), jnp.float32)

            @plsc.parallel_loop(0, nchunk, step=1, unroll=8,
                                carry=(z, z, z, z))
            def chunk_body(j, acc4):
                a0, a1, a2, a3 = acc4
                o = j * _L
                vc = v_vmem[pl.ds(o, _L)]
                a0 = a0 + buf[slot, 0, pl.ds(o, _L)] * vc
                a1 = a1 + buf[slot, 1, pl.ds(o, _L)] * vc
                a2 = a2 + buf[slot, 2, pl.ds(o, _L)] * vc
                a3 = a3 + buf[slot, 3, pl.ds(o, _L)] * vc
                return a0, a1, a2, a3

            accs.extend(chunk_body)
        lane_ids = jax.lax.iota(jnp.int32, _L)
        res = jnp.zeros((_L,), jnp.float32)
        for r, acc in enumerate(accs):
            res = jnp.where(lane_ids == r, jnp.sum(acc), res)
        outv[pl.ds(sb * _L, _L)] = res
        return carry

    jax.lax.fori_loop(0, nsuper, super_body, 0)

    if _XTRA:
        @pl.when(wid < _XTRA)
        def _():
            pltpu.sync_copy(outv, out_hbm.at[pl.ds(obase, _MAXR)])

        @pl.when(wid >= _XTRA)
        def _():
            pltpu.sync_copy(outv.at[pl.ds(0, _L * _BSUP)],
                            out_hbm.at[pl.ds(obase, _L * _BSUP)])
    else:
        pltpu.sync_copy(outv, out_hbm.at[pl.ds(obase, _MAXR)])


_sc_matvec = pl.kernel(
    _sc_matvec_body,
    out_type=jax.ShapeDtypeStruct((_NSC,), jnp.float32),
    mesh=plsc.VectorSubcoreMesh(core_axis_name="c", subcore_axis_name="s"),
    compiler_params=pltpu.CompilerParams(needs_layout_passes=False),
    scratch_types=[
        pltpu.VMEM((_N,), jnp.float32),
        pltpu.VMEM((2, _SUB, _N), jnp.float32),
        pltpu.VMEM((_MAXR,), jnp.float32),
        pltpu.SemaphoreType.DMA,
        pltpu.SemaphoreType.DMA,
    ],
)


def kernel(x, adj, W, b):
    n, d = x.shape
    c = W.shape[1]
    vrow = pl.pallas_call(
        _v_row_kernel,
        out_shape=jax.ShapeDtypeStruct((1, n), jnp.float32),
    )(x, W.T, b.reshape(1, c))

    out_tc = pl.pallas_call(
        _tc_matvec_kernel,
        grid=(_S0 // _BN_TC,),
        in_specs=[
            pl.BlockSpec((_BN_TC, n), lambda i: (i, 0)),
            pl.BlockSpec((1, n), lambda i: (0, 0)),
        ],
        out_specs=pl.BlockSpec((_BN_TC, 1), lambda i: (i, 0)),
        out_shape=jax.ShapeDtypeStruct((_S0, 1), jnp.float32),
    )(adj, vrow)

    out_sc = _sc_matvec(adj, vrow.reshape(n))
    return jnp.concatenate([out_tc[:, 0], out_sc])
